# Initial kernel scaffold; baseline (speedup 1.0000x reference)
import jax
import jax.numpy as jnp
from jax.experimental import pallas as pl


def kernel(slot_vals, slot_lengths, embed_table, W, b):
    raise NotImplementedError("write your pallas kernel here")


# speedup vs baseline: 9.7782x; 9.7782x over previous
import functools

import jax
import jax.numpy as jnp
from jax import lax
from jax.experimental import pallas as pl
from jax.experimental.pallas import tpu as pltpu
from jax.experimental.pallas import tpu_sc as plsc

_B, _S, _L, _H, _V = 1024, 26, 20, 128, 100000
_M = _B * _S
_NC, _NS = 2, 16
_NW = _NC * _NS
_PW = _M // _NW
_CH = 64
_NCH = _PW // _CH

_mesh = plsc.VectorSubcoreMesh(core_axis_name="c", subcore_axis_name="s")


@functools.partial(
    pl.kernel,
    out_type=jax.ShapeDtypeStruct((_M, _H), jnp.float32),
    mesh=_mesh,
    scratch_types=[
        pltpu.VMEM((_L, _CH), jnp.int32),
        pltpu.VMEM((_PW,), jnp.int32),
        pltpu.VMEM((_NCH * _L, _CH), jnp.int32),
        pltpu.VMEM((_PW, _H), jnp.float32),
        pltpu.SemaphoreType.DMA,
        pltpu.SemaphoreType.DMA((_NCH,)),
    ],
    compiler_params=pltpu.CompilerParams(use_tc_tiling_on_sc=False),
)
def _sc_pool(vals_hbm, lens_hbm, table_hbm, out_hbm,
             valsk_v, lens_v, idx_v, tile_acc, ld_sem, sems):
    cid = lax.axis_index("c")
    sid = lax.axis_index("s")
    wid = sid * _NC + cid
    base = wid * _PW

    pltpu.sync_copy(lens_hbm.at[pl.ds(base, _PW)], lens_v)

    def prep_l(l, k):
        r = k * _L + l
        lvec = jnp.zeros((16,), jnp.int32) + l
        fill = jnp.where(lvec == 0, 0, -1)
        for v in range(_CH // 16):
            vals16 = valsk_v[l, pl.ds(v * 16, 16)]
            lens16 = lens_v[pl.ds(k * _CH + v * 16, 16)]
            idx_v[r, pl.ds(v * 16, 16)] = jnp.where(lvec < lens16, vals16, fill)
        return k

    def prep_chain(k, c):
        pltpu.async_copy(
            vals_hbm.at[:, pl.ds(base + k * _CH, _CH)], valsk_v, ld_sem).wait()
        lax.fori_loop(0, _L, prep_l, k)
        return c

    lax.fori_loop(0, _NCH, prep_chain, 0)
    descs = [None] * _NCH
    for l in range(_L):
        for k in range(_NCH):
            if l > 0:
                descs[k].wait()
            r = k * _L + l
            dst = tile_acc.at[pl.ds(k * _CH, _CH)]
            if l == 0:
                descs[k] = pltpu.async_copy(
                    table_hbm.at[idx_v.at[r]], dst, sems.at[k])
            else:
                descs[k] = pltpu.async_copy(
                    table_hbm.at[plsc.Indices(idx_v.at[r], ignored_value=-1)],
                    dst, sems.at[k], add=True)
    for k in range(_NCH):
        descs[k].wait()

    pltpu.sync_copy(tile_acc, out_hbm.at[pl.ds(base, _PW)])


def _tc_body(pooled_ref, lens_ref, w_ref, b_ref, t0_ref, out_ref):
    x = pooled_ref[...]
    lenf = lens_ref[...].astype(jnp.float32)
    scale = 1.0 / (lenf + 1e-5)
    xw = lax.dot_general(x, w_ref[...], (((1,), (1,)), ((), ())),
                         preferred_element_type=jnp.float32)
    t0w = lax.dot_general(t0_ref[...], w_ref[...], (((1,), (1,)), ((), ())),
                          preferred_element_type=jnp.float32)
    empty = (lenf == 0.0).astype(jnp.float32)
    out_ref[...] = (xw - empty * t0w + lenf * b_ref[...]) * scale


_BM = 1024


def _tc_linear(pooled, lens2, W, b2, t0):
    return pl.pallas_call(
        _tc_body,
        grid=(_M // _BM,),
        in_specs=[
            pl.BlockSpec((_BM, _H), lambda i: (i, 0)),
            pl.BlockSpec((_BM, 1), lambda i: (i, 0)),
            pl.BlockSpec((_H, _H), lambda i: (0, 0)),
            pl.BlockSpec((1, _H), lambda i: (0, 0)),
            pl.BlockSpec((1, _H), lambda i: (0, 0)),
        ],
        out_specs=pl.BlockSpec((_BM, _H), lambda i: (i, 0)),
        out_shape=jax.ShapeDtypeStruct((_M, _H), jnp.float32),
    )(pooled, lens2, W, b2, t0)


def kernel(slot_vals, slot_lengths, embed_table, W, b):
    vals_t = slot_vals.reshape(_M, _L).T
    lens1 = slot_lengths.reshape(_M)
    pooled = _sc_pool(vals_t, lens1, embed_table)
    out = _tc_linear(pooled, lens1.reshape(_M, 1).astype(jnp.int32),
                     W, b.reshape(1, _H), embed_table[0:1])
    return out.reshape(_B, _S, _H)
